# baseline (speedup 1.0000x reference)
import functools

import jax
import jax.numpy as jnp
from jax import lax
from jax.experimental import pallas as pl
from jax.experimental.pallas import tpu as pltpu
from jax.experimental.pallas import tpu_sc as plsc

EMB_D = 32
CB_K = 1024
BLOCK_N = 1024

SC_NC = 2
SC_NS = 16
SC_NW = SC_NC * SC_NS
SC_CHUNK = 128


def _vq_block(x_ref, emb_ref, enc_ref, idx_ref, loss_ref):
    x = x_ref[...]
    emb = emb_ref[...]
    sim = jax.lax.dot_general(
        x, emb, (((1,), (0,)), ((), ())),
        preferred_element_type=jnp.float32)
    n2 = jnp.sum(x * x, axis=1, keepdims=True)
    c2 = jnp.sum(emb * emb, axis=0, keepdims=True)
    d = (n2 - 2.0 * sim) + c2
    m = jnp.min(d, axis=1, keepdims=True)
    kiota = jax.lax.broadcasted_iota(jnp.int32, d.shape, 1)
    idx = jnp.min(jnp.where(d == m, kiota, CB_K), axis=1)
    enc_ref[...] = jnp.where(kiota == idx[:, None], 1.0, 0.0).astype(jnp.float32)
    idx_ref[...] = idx[None, :]
    part = jnp.sum(m).reshape(1, 1)

    @pl.when(pl.program_id(0) == 0)
    def _init():
        loss_ref[...] = jnp.zeros((1, 1), jnp.float32)

    loss_ref[...] += part


def _sc_gather_body(table_hbm, idx_hbm, out_hbm, idx_v, rows_v, sem):
    wid = lax.axis_index("s") * SC_NC + lax.axis_index("c")
    nchunk = idx_v.shape[0]
    b_per_w = nchunk * SC_CHUNK
    base = wid * b_per_w
    pltpu.sync_copy(idx_hbm.at[wid], idx_v)
    copies = []
    for j in range(nchunk):
        copies.append(pltpu.async_copy(
            table_hbm.at[idx_v.at[j]],
            rows_v.at[pl.ds(j * SC_CHUNK, SC_CHUNK)],
            sem))
    for cp in copies:
        cp.wait()
    pltpu.sync_copy(rows_v, out_hbm.at[pl.ds(base, b_per_w)])


def kernel(inputs, embeddings):
    n = inputs.shape[0] * inputs.shape[1]
    x = inputs.reshape(n, EMB_D)
    grid = n // BLOCK_N
    enc, idx2d, loss = pl.pallas_call(
        _vq_block,
        grid=(grid,),
        in_specs=[
            pl.BlockSpec((BLOCK_N, EMB_D), lambda i: (i, 0)),
            pl.BlockSpec((EMB_D, CB_K), lambda i: (0, 0)),
        ],
        out_specs=[
            pl.BlockSpec((BLOCK_N, CB_K), lambda i: (i, 0)),
            pl.BlockSpec((1, BLOCK_N), lambda i: (0, i)),
            pl.BlockSpec((1, 1), lambda i: (0, 0)),
        ],
        out_shape=[
            jax.ShapeDtypeStruct((n, CB_K), jnp.float32),
            jax.ShapeDtypeStruct((1, n), jnp.int32),
            jax.ShapeDtypeStruct((1, 1), jnp.float32),
        ],
    )(x, embeddings)

    b_per_w = n // SC_NW
    nchunk = b_per_w // SC_CHUNK
    emb_t = embeddings.T
    idx_sc = idx2d.reshape(SC_NW, nchunk, SC_CHUNK)
    q = pl.kernel(
        _sc_gather_body,
        out_type=jax.ShapeDtypeStruct((n, EMB_D), jnp.float32),
        mesh=plsc.VectorSubcoreMesh(core_axis_name="c", subcore_axis_name="s"),
        scratch_types=[
            pltpu.VMEM((nchunk, SC_CHUNK), jnp.int32),
            pltpu.VMEM((b_per_w, EMB_D), jnp.float32),
            pltpu.SemaphoreType.DMA,
        ],
        compiler_params=pltpu.CompilerParams(use_tc_tiling_on_sc=False),
    )(emb_t, idx_sc)

    quantized_st = q.reshape(inputs.shape)
    indices_r = idx2d.reshape(inputs.shape[:-1])
    commitment_loss = (loss[0, 0] / (n * EMB_D)).astype(jnp.float32)
    return quantized_st, enc, indices_r, commitment_loss


# speedup vs baseline: 1.0890x; 1.0890x over previous
import functools

import jax
import jax.numpy as jnp
from jax import lax
from jax.experimental import pallas as pl
from jax.experimental.pallas import tpu as pltpu
from jax.experimental.pallas import tpu_sc as plsc

EMB_D = 32
CB_K = 1024
BLOCK_N = 4096

SC_NC = 2
SC_NS = 16
SC_NW = SC_NC * SC_NS
SC_CHUNK = 128


def _vq_block(x_ref, emb_ref, enc_ref, idx_ref, loss_ref):
    x = x_ref[...]
    emb = emb_ref[...]
    sim = jax.lax.dot_general(
        x, emb, (((1,), (0,)), ((), ())),
        preferred_element_type=jnp.float32)
    n2 = jnp.sum(x * x, axis=1, keepdims=True)
    c2 = jnp.sum(emb * emb, axis=0, keepdims=True)
    d = (n2 - 2.0 * sim) + c2
    m = jnp.min(d, axis=1, keepdims=True)
    kiota = jax.lax.broadcasted_iota(jnp.int32, d.shape, 1)
    idx = jnp.min(jnp.where(d == m, kiota, CB_K), axis=1)
    enc_ref[...] = jnp.where(kiota == idx[:, None], 1.0, 0.0).astype(jnp.float32)
    idx_ref[...] = idx[None, :]
    part = jnp.sum(m).reshape(1, 1)

    @pl.when(pl.program_id(0) == 0)
    def _init():
        loss_ref[...] = jnp.zeros((1, 1), jnp.float32)

    loss_ref[...] += part


def _sc_gather_body(table_hbm, idx_hbm, out_hbm, idx_v, rows_v, sem):
    wid = lax.axis_index("s") * SC_NC + lax.axis_index("c")
    nchunk = idx_v.shape[0]
    b_per_w = nchunk * SC_CHUNK
    base = wid * b_per_w
    pltpu.sync_copy(idx_hbm.at[wid], idx_v)
    copies = []
    for j in range(nchunk):
        copies.append(pltpu.async_copy(
            table_hbm.at[idx_v.at[j]],
            rows_v.at[pl.ds(j * SC_CHUNK, SC_CHUNK)],
            sem))
    for cp in copies:
        cp.wait()
    pltpu.sync_copy(rows_v, out_hbm.at[pl.ds(base, b_per_w)])


def kernel(inputs, embeddings):
    n = inputs.shape[0] * inputs.shape[1]
    x = inputs.reshape(n, EMB_D)
    grid = n // BLOCK_N
    enc, idx2d, loss = pl.pallas_call(
        _vq_block,
        grid=(grid,),
        in_specs=[
            pl.BlockSpec((BLOCK_N, EMB_D), lambda i: (i, 0)),
            pl.BlockSpec((EMB_D, CB_K), lambda i: (0, 0)),
        ],
        out_specs=[
            pl.BlockSpec((BLOCK_N, CB_K), lambda i: (i, 0)),
            pl.BlockSpec((1, BLOCK_N), lambda i: (0, i)),
            pl.BlockSpec((1, 1), lambda i: (0, 0)),
        ],
        out_shape=[
            jax.ShapeDtypeStruct((n, CB_K), jnp.float32),
            jax.ShapeDtypeStruct((1, n), jnp.int32),
            jax.ShapeDtypeStruct((1, 1), jnp.float32),
        ],
    )(x, embeddings)

    b_per_w = n // SC_NW
    nchunk = b_per_w // SC_CHUNK
    emb_t = embeddings.T
    idx_sc = idx2d.reshape(SC_NW, nchunk, SC_CHUNK)
    q = pl.kernel(
        _sc_gather_body,
        out_type=jax.ShapeDtypeStruct((n, EMB_D), jnp.float32),
        mesh=plsc.VectorSubcoreMesh(core_axis_name="c", subcore_axis_name="s"),
        scratch_types=[
            pltpu.VMEM((nchunk, SC_CHUNK), jnp.int32),
            pltpu.VMEM((b_per_w, EMB_D), jnp.float32),
            pltpu.SemaphoreType.DMA,
        ],
        compiler_params=pltpu.CompilerParams(use_tc_tiling_on_sc=False),
    )(emb_t, idx_sc)

    quantized_st = q.reshape(inputs.shape)
    indices_r = idx2d.reshape(inputs.shape[:-1])
    commitment_loss = (loss[0, 0] / (n * EMB_D)).astype(jnp.float32)
    return quantized_st, enc, indices_r, commitment_loss


# speedup vs baseline: 1.1097x; 1.0190x over previous
import functools

import jax
import jax.numpy as jnp
from jax import lax
from jax.experimental import pallas as pl
from jax.experimental.pallas import tpu as pltpu
from jax.experimental.pallas import tpu_sc as plsc

EMB_D = 32
CB_K = 1024
BLOCK_N = 4096

SC_NC = 2
SC_NS = 16
SC_NW = SC_NC * SC_NS
SC_CHUNK = 128


def _vq_block(x_ref, emb_ref, enc_ref, idx_ref, loss_ref):
    x = x_ref[...]
    emb = emb_ref[...]
    msim2 = jax.lax.dot_general(
        x, emb * -2.0, (((1,), (0,)), ((), ())),
        preferred_element_type=jnp.float32)
    n2 = jnp.sum(x * x, axis=1, keepdims=True)
    c2 = jnp.sum(emb * emb, axis=0, keepdims=True)
    d = (n2 + msim2) + c2
    m = jnp.min(d, axis=1, keepdims=True)
    kiota = jax.lax.broadcasted_iota(jnp.int32, d.shape, 1)
    idx = jnp.min(jnp.where(d == m, kiota, CB_K), axis=1)
    enc_ref[...] = jnp.where(kiota == idx[:, None], 1.0, 0.0).astype(jnp.float32)
    idx_ref[...] = idx[None, :]
    part = jnp.sum(m).reshape(1, 1)

    @pl.when(pl.program_id(0) == 0)
    def _init():
        loss_ref[...] = jnp.zeros((1, 1), jnp.float32)

    loss_ref[...] += part


def _sc_gather_body(table_hbm, idx_hbm, out_hbm, idx_v, rows_v, sem):
    wid = lax.axis_index("s") * SC_NC + lax.axis_index("c")
    nchunk = idx_v.shape[0]
    b_per_w = nchunk * SC_CHUNK
    base = wid * b_per_w
    pltpu.sync_copy(idx_hbm.at[wid], idx_v)
    copies = []
    for j in range(nchunk):
        copies.append(pltpu.async_copy(
            table_hbm.at[idx_v.at[j]],
            rows_v.at[pl.ds(j * SC_CHUNK, SC_CHUNK)],
            sem))
    for cp in copies:
        cp.wait()
    pltpu.sync_copy(rows_v, out_hbm.at[pl.ds(base, b_per_w)])


def kernel(inputs, embeddings):
    n = inputs.shape[0] * inputs.shape[1]
    x = inputs.reshape(n, EMB_D)
    grid = n // BLOCK_N
    enc, idx2d, loss = pl.pallas_call(
        _vq_block,
        grid=(grid,),
        in_specs=[
            pl.BlockSpec((BLOCK_N, EMB_D), lambda i: (i, 0)),
            pl.BlockSpec((EMB_D, CB_K), lambda i: (0, 0)),
        ],
        out_specs=[
            pl.BlockSpec((BLOCK_N, CB_K), lambda i: (i, 0)),
            pl.BlockSpec((1, BLOCK_N), lambda i: (0, i)),
            pl.BlockSpec((1, 1), lambda i: (0, 0)),
        ],
        out_shape=[
            jax.ShapeDtypeStruct((n, CB_K), jnp.float32),
            jax.ShapeDtypeStruct((1, n), jnp.int32),
            jax.ShapeDtypeStruct((1, 1), jnp.float32),
        ],
    )(x, embeddings)

    b_per_w = n // SC_NW
    nchunk = b_per_w // SC_CHUNK
    emb_t = embeddings.T
    idx_sc = idx2d.reshape(SC_NW, nchunk, SC_CHUNK)
    q = pl.kernel(
        _sc_gather_body,
        out_type=jax.ShapeDtypeStruct((n, EMB_D), jnp.float32),
        mesh=plsc.VectorSubcoreMesh(core_axis_name="c", subcore_axis_name="s"),
        scratch_types=[
            pltpu.VMEM((nchunk, SC_CHUNK), jnp.int32),
            pltpu.VMEM((b_per_w, EMB_D), jnp.float32),
            pltpu.SemaphoreType.DMA,
        ],
        compiler_params=pltpu.CompilerParams(use_tc_tiling_on_sc=False),
    )(emb_t, idx_sc)

    quantized_st = q.reshape(inputs.shape)
    indices_r = idx2d.reshape(inputs.shape[:-1])
    commitment_loss = (loss[0, 0] / (n * EMB_D)).astype(jnp.float32)
    return quantized_st, enc, indices_r, commitment_loss


# speedup vs baseline: 1.1307x; 1.0189x over previous
import functools

import jax
import jax.numpy as jnp
from jax import lax
from jax.experimental import pallas as pl
from jax.experimental.pallas import tpu as pltpu
from jax.experimental.pallas import tpu_sc as plsc

EMB_D = 32
CB_K = 1024
BLOCK_N = 4096

SC_NC = 2
SC_NS = 16
SC_NW = SC_NC * SC_NS
SC_CHUNK = 128


def _vq_block(x_ref, emb_ref, enc_ref, idx_ref, loss_ref):
    x = x_ref[...]
    emb = emb_ref[...]
    msim2 = jax.lax.dot_general(
        x, emb * -2.0, (((1,), (0,)), ((), ())),
        preferred_element_type=jnp.float32)
    n2 = jnp.sum(x * x, axis=1, keepdims=True)
    c2 = jnp.sum(emb * emb, axis=0, keepdims=True)
    d = (n2 + msim2) + c2
    m = jnp.min(d, axis=1, keepdims=True)
    kiota = jax.lax.broadcasted_iota(jnp.int32, d.shape, 1)
    idx = jnp.min(jnp.where(d == m, kiota, CB_K), axis=1)
    enc_ref[...] = jnp.where(kiota == idx[:, None], 1.0, 0.0).astype(jnp.float32)
    idx_ref[...] = idx[None, :]
    loss_ref[...] = jnp.sum(m).reshape(1, 1, 1)


def _sc_gather_body(table_hbm, idx_hbm, out_hbm, idx_v, rows_v, sem):
    wid = lax.axis_index("s") * SC_NC + lax.axis_index("c")
    nchunk = idx_v.shape[0]
    b_per_w = nchunk * SC_CHUNK
    base = wid * b_per_w
    pltpu.sync_copy(idx_hbm.at[wid], idx_v)
    copies = []
    for j in range(nchunk):
        copies.append(pltpu.async_copy(
            table_hbm.at[idx_v.at[j]],
            rows_v.at[pl.ds(j * SC_CHUNK, SC_CHUNK)],
            sem))
    for cp in copies:
        cp.wait()
    pltpu.sync_copy(rows_v, out_hbm.at[pl.ds(base, b_per_w)])


def kernel(inputs, embeddings):
    n = inputs.shape[0] * inputs.shape[1]
    x = inputs.reshape(n, EMB_D)
    grid = n // BLOCK_N
    enc, idx2d, loss = pl.pallas_call(
        _vq_block,
        grid=(grid,),
        in_specs=[
            pl.BlockSpec((BLOCK_N, EMB_D), lambda i: (i, 0)),
            pl.BlockSpec((EMB_D, CB_K), lambda i: (0, 0)),
        ],
        out_specs=[
            pl.BlockSpec((BLOCK_N, CB_K), lambda i: (i, 0)),
            pl.BlockSpec((1, BLOCK_N), lambda i: (0, i)),
            pl.BlockSpec((1, 1, 1), lambda i: (i, 0, 0)),
        ],
        out_shape=[
            jax.ShapeDtypeStruct((n, CB_K), jnp.float32),
            jax.ShapeDtypeStruct((1, n), jnp.int32),
            jax.ShapeDtypeStruct((grid, 1, 1), jnp.float32),
        ],
        compiler_params=pltpu.CompilerParams(
            dimension_semantics=("parallel",)),
    )(x, embeddings)

    b_per_w = n // SC_NW
    nchunk = b_per_w // SC_CHUNK
    emb_t = embeddings.T
    idx_sc = idx2d.reshape(SC_NW, nchunk, SC_CHUNK)
    q = pl.kernel(
        _sc_gather_body,
        out_type=jax.ShapeDtypeStruct((n, EMB_D), jnp.float32),
        mesh=plsc.VectorSubcoreMesh(core_axis_name="c", subcore_axis_name="s"),
        scratch_types=[
            pltpu.VMEM((nchunk, SC_CHUNK), jnp.int32),
            pltpu.VMEM((b_per_w, EMB_D), jnp.float32),
            pltpu.SemaphoreType.DMA,
        ],
        compiler_params=pltpu.CompilerParams(use_tc_tiling_on_sc=False),
    )(emb_t, idx_sc)

    quantized_st = q.reshape(inputs.shape)
    indices_r = idx2d.reshape(inputs.shape[:-1])
    commitment_loss = (jnp.sum(loss) / (n * EMB_D)).astype(jnp.float32)
    return quantized_st, enc, indices_r, commitment_loss


# speedup vs baseline: 1.1592x; 1.0252x over previous
import jax
import jax.numpy as jnp
from jax import lax
from jax.experimental import pallas as pl
from jax.experimental.pallas import tpu as pltpu
from jax.experimental.pallas import tpu_sc as plsc

EMB_D = 32
CB_K = 1024
BLOCK_B = 4
SUB_N = 1024

SC_NC = 2
SC_NS = 16
SC_NW = SC_NC * SC_NS
SC_CHUNK = 128


def _vq_block(x_ref, emb_ref, enc_ref, idx_ref, loss_ref):
    emb = emb_ref[...]
    memb2 = emb * -2.0
    c2 = jnp.sum(emb * emb, axis=0, keepdims=True)
    loss = jnp.zeros((), jnp.float32)
    for j in range(BLOCK_B):
        x = x_ref[j]
        msim2 = jax.lax.dot_general(
            x, memb2, (((1,), (0,)), ((), ())),
            preferred_element_type=jnp.float32)
        n2 = jnp.sum(x * x, axis=1, keepdims=True)
        d = (n2 + msim2) + c2
        m = jnp.min(d, axis=1, keepdims=True)
        kiota = jax.lax.broadcasted_iota(jnp.int32, d.shape, 1)
        idx = jnp.min(jnp.where(d == m, kiota, CB_K), axis=1)
        enc_ref[pl.ds(j * SUB_N, SUB_N), :] = jnp.where(
            kiota == idx[:, None], 1.0, 0.0).astype(jnp.float32)
        idx_ref[0, j, :] = idx
        loss = loss + jnp.sum(m)
    loss_ref[...] = loss.reshape(1, 1, 1)


def _sc_gather_body(table_hbm, idx_hbm, out_hbm, idx_v, rows_v, sem):
    wid = lax.axis_index("s") * SC_NC + lax.axis_index("c")
    nchunk = idx_v.shape[0]
    pltpu.sync_copy(idx_hbm.at[wid], idx_v)
    copies = []
    for j in range(nchunk):
        copies.append(pltpu.async_copy(
            table_hbm.at[idx_v.at[j]],
            rows_v.at[pl.ds(j * SC_CHUNK, SC_CHUNK)],
            sem))
    for cp in copies:
        cp.wait()
    pltpu.sync_copy(rows_v, out_hbm.at[wid])


def kernel(inputs, embeddings):
    nb, sub_n, _ = inputs.shape
    n = nb * sub_n
    grid = nb // BLOCK_B
    enc, idx3, loss = pl.pallas_call(
        _vq_block,
        grid=(grid,),
        in_specs=[
            pl.BlockSpec((BLOCK_B, SUB_N, EMB_D), lambda i: (i, 0, 0)),
            pl.BlockSpec((EMB_D, CB_K), lambda i: (0, 0)),
        ],
        out_specs=[
            pl.BlockSpec((BLOCK_B * SUB_N, CB_K), lambda i: (i, 0)),
            pl.BlockSpec((1, BLOCK_B, SUB_N), lambda i: (i, 0, 0)),
            pl.BlockSpec((1, 1, 1), lambda i: (i, 0, 0)),
        ],
        out_shape=[
            jax.ShapeDtypeStruct((n, CB_K), jnp.float32),
            jax.ShapeDtypeStruct((grid, BLOCK_B, SUB_N), jnp.int32),
            jax.ShapeDtypeStruct((grid, 1, 1), jnp.float32),
        ],
        compiler_params=pltpu.CompilerParams(
            dimension_semantics=("parallel",)),
    )(inputs, embeddings)

    idx2d = idx3.reshape(nb, sub_n)
    nchunk = sub_n // SC_CHUNK
    emb_t = embeddings.T
    idx_sc = idx2d.reshape(SC_NW, nchunk, SC_CHUNK)
    quantized_st = pl.kernel(
        _sc_gather_body,
        out_type=jax.ShapeDtypeStruct((nb, sub_n, EMB_D), jnp.float32),
        mesh=plsc.VectorSubcoreMesh(core_axis_name="c", subcore_axis_name="s"),
        scratch_types=[
            pltpu.VMEM((nchunk, SC_CHUNK), jnp.int32),
            pltpu.VMEM((sub_n, EMB_D), jnp.float32),
            pltpu.SemaphoreType.DMA,
        ],
        compiler_params=pltpu.CompilerParams(use_tc_tiling_on_sc=False),
    )(emb_t, idx_sc)

    commitment_loss = (jnp.sum(loss) / (n * EMB_D)).astype(jnp.float32)
    return quantized_st, enc, idx2d, commitment_loss
